# baseline (speedup 1.0000x reference)
import jax
import jax.numpy as jnp
from jax import lax
from jax.experimental import pallas as pl
from jax.experimental.pallas import tpu as pltpu

_B = 512
_C = 512
_T = 128
_BB = 8


def _tc_body(x_ref, o_ref):
    blk = x_ref[...]
    rows = lax.broadcasted_iota(jnp.int32, (_BB, _T, _T), 1)
    cols = lax.broadcasted_iota(jnp.int32, (_BB, _T, _T), 2)
    masked = jnp.where(rows == cols, blk, 0.0)
    ones = jnp.ones((_T,), jnp.float32)
    o_ref[...] = lax.dot_general(
        masked, ones, (((1,), (0,)), ((), ())),
        preferred_element_type=jnp.float32)


@jax.jit
def _diag_tc(x):
    return pl.pallas_call(
        _tc_body,
        grid=(_B // _BB, _C // _T),
        in_specs=[pl.BlockSpec((_BB, _T, _T), lambda b, u: (b, u, u))],
        out_specs=pl.BlockSpec((_BB, _T), lambda b, u: (b, u)),
        out_shape=jax.ShapeDtypeStruct((_B, _C), jnp.float32),
        compiler_params=pltpu.CompilerParams(
            dimension_semantics=("arbitrary", "arbitrary")),
    )(x)


def kernel(x):
    return _diag_tc(x)


# speedup vs baseline: 1.9125x; 1.9125x over previous
import functools

import jax
import jax.numpy as jnp
from jax import lax
from jax.experimental import pallas as pl
from jax.experimental.pallas import tpu as pltpu
from jax.experimental.pallas import tpu_sc as plsc

_B = 512
_C = 512
_L = 16
_T = 128
_NW = 32
_B_SC = 320
_B_PER_W = _B_SC // _NW
_BLK_PER_B = _C // _T
_RB = 2
_NROUND = _B_PER_W * _BLK_PER_B // _RB
_BB_TC = 8


def _diag_sc_body(x_hbm, out_hbm, buf_v, out_v, sem0, sem1):
    cid = lax.axis_index("c")
    sid = lax.axis_index("s")
    wid = sid * 2 + cid
    lanes = lax.iota(jnp.int32, 16)
    sems = [sem0, sem1]

    def block_refs(r, parity, k):
        g = (wid * _NROUND + r) * _RB + k
        row0 = pl.multiple_of(g * _T, _T)
        col0 = pl.multiple_of((g % _BLK_PER_B) * _T, _T)
        return x_hbm.at[pl.ds(row0, _T), pl.ds(col0, _T)], buf_v.at[parity, k]

    def fire(r, parity):
        for k in range(_RB):
            src, dst = block_refs(r, parity, k)
            pltpu.async_copy(src, dst, sems[parity])

    def drain_extract(r, parity, i, j0):
        for k in range(_RB):
            src, dst = block_refs(r, parity, k)
            pltpu.make_async_copy(src, dst, sems[parity]).wait()
        for k in range(_RB):
            for q in range(_T // _L):
                d = q * _L + lanes
                diag = plsc.load_gather(buf_v.at[parity, k], [d, d])
                out_v[pl.ds(i * _C + (j0 + k) * _T + q * _L, _L)] = diag

    def per_super(s, carry):
        fire(2 * s + 1, 1)
        drain_extract(2 * s, 0, s, 0)

        @pl.when(s + 1 < _NROUND // 2)
        def _():
            fire(2 * s + 2, 0)

        drain_extract(2 * s + 1, 1, s, 2)
        return carry

    fire(0, 0)
    lax.fori_loop(0, _NROUND // 2, per_super, 0)
    pltpu.sync_copy(out_v, out_hbm.at[wid])


def _diag_tc_body(x_ref, eye_ref, o_ref):
    blk = x_ref[...]
    masked = blk * eye_ref[...]
    ones = jnp.ones((_T,), jnp.float32)
    o_ref[...] = lax.dot_general(
        masked, ones, (((1,), (0,)), ((), ())),
        preferred_element_type=jnp.float32)


@jax.jit
def _diag(x):
    x2d = x.reshape(_B * _C, _C)
    mesh = plsc.VectorSubcoreMesh(core_axis_name="c", subcore_axis_name="s")
    sc = functools.partial(
        pl.kernel,
        mesh=mesh,
        out_type=jax.ShapeDtypeStruct((_NW, _B_PER_W * _C), jnp.float32),
        scratch_types=[
            pltpu.VMEM((2, _RB, _T, _T), jnp.float32),
            pltpu.VMEM((_B_PER_W * _C,), jnp.float32),
            pltpu.SemaphoreType.DMA,
            pltpu.SemaphoreType.DMA,
        ],
        compiler_params=pltpu.CompilerParams(needs_layout_passes=False),
    )(_diag_sc_body)
    out_sc = sc(x2d).reshape(_B_SC, _C)

    eye = jnp.eye(_T, dtype=jnp.float32)[None]
    out_tc = pl.pallas_call(
        _diag_tc_body,
        grid=((_B - _B_SC) // _BB_TC, _C // _T),
        in_specs=[
            pl.BlockSpec((_BB_TC, _T, _T),
                         lambda b, u: (b + _B_SC // _BB_TC, u, u)),
            pl.BlockSpec((1, _T, _T), lambda b, u: (0, 0, 0)),
        ],
        out_specs=pl.BlockSpec((_BB_TC, _T), lambda b, u: (b, u)),
        out_shape=jax.ShapeDtypeStruct((_B - _B_SC, _C), jnp.float32),
        compiler_params=pltpu.CompilerParams(
            dimension_semantics=("arbitrary", "arbitrary")),
    )(x, eye)

    return jnp.concatenate([out_sc, out_tc], axis=0)


def kernel(x):
    B, C, C1 = x.shape
    assert (B, C, C1) == (_B, _C, _C)
    return _diag(x)


# speedup vs baseline: 2.5192x; 1.3172x over previous
import functools

import jax
import jax.numpy as jnp
from jax import lax
from jax.experimental import pallas as pl
from jax.experimental.pallas import tpu as pltpu
from jax.experimental.pallas import tpu_sc as plsc

_B = 512
_C = 512
_L = 16
_T = 128
_NW = 32
_B_PER_W = _B // _NW
_BLK_PER_B = _C // _T
_BLK_PER_W = _B_PER_W * _BLK_PER_B
_RB = 2
_NROUND = _BLK_PER_W // _RB


def _diag_body(x_hbm, out_hbm, buf_v, out_v, sem0, sem1):
    cid = lax.axis_index("c")
    sid = lax.axis_index("s")
    wid = sid * 2 + cid
    lanes = lax.iota(jnp.int32, 16)
    sems = [sem0, sem1]

    def block_refs(r, parity, k):
        g = (wid * _NROUND + r) * _RB + k
        row0 = pl.multiple_of(g * _T, _T)
        col0 = pl.multiple_of((g % _BLK_PER_B) * _T, _T)
        return x_hbm.at[pl.ds(row0, _T), pl.ds(col0, _T)], buf_v.at[parity, k]

    def fire(r, parity):
        for k in range(_RB):
            src, dst = block_refs(r, parity, k)
            pltpu.async_copy(src, dst, sems[parity])

    def drain_extract(r, parity, i, j0):
        for k in range(_RB):
            src, dst = block_refs(r, parity, k)
            pltpu.make_async_copy(src, dst, sems[parity]).wait()
        for k in range(_RB):
            for q in range(_T // _L):
                d = q * _L + lanes
                diag = plsc.load_gather(buf_v.at[parity, k], [d, d])
                out_v[i, pl.ds((j0 + k) * _T + q * _L, _L)] = diag

    def per_super(s, carry):
        fire(2 * s + 1, 1)
        drain_extract(2 * s, 0, s, 0)

        @pl.when(s + 1 < _NROUND // 2)
        def _():
            fire(2 * s + 2, 0)

        drain_extract(2 * s + 1, 1, s, 2)
        return carry

    fire(0, 0)
    lax.fori_loop(0, _NROUND // 2, per_super, 0)
    pltpu.sync_copy(out_v, out_hbm.at[pl.ds(wid * _B_PER_W, _B_PER_W)])


@jax.jit
def _diag(x2d):
    mesh = plsc.VectorSubcoreMesh(core_axis_name="c", subcore_axis_name="s")
    f = functools.partial(
        pl.kernel,
        mesh=mesh,
        out_type=jax.ShapeDtypeStruct((_B, _C), jnp.float32),
        scratch_types=[
            pltpu.VMEM((2, _RB, _T, _T), jnp.float32),
            pltpu.VMEM((_B_PER_W, _C), jnp.float32),
            pltpu.SemaphoreType.DMA,
            pltpu.SemaphoreType.DMA,
        ],
        compiler_params=pltpu.CompilerParams(needs_layout_passes=False),
    )(_diag_body)
    return f(x2d)


def kernel(x):
    B, C, C1 = x.shape
    assert (B, C, C1) == (_B, _C, _C)
    return _diag(x.reshape(_B * _C, _C))
